# baseline (speedup 1.0000x reference)
import functools

import jax
import jax.numpy as jnp
from jax import lax
from jax.experimental import pallas as pl
from jax.experimental.pallas import tpu as pltpu
from jax.experimental.pallas import tpu_sc as plsc

_B = 16384
_L = 50
_D_ID = 64
_N_STU = 1000000
_NC = 2
_NS = 16
_NW = _NC * _NS
_ROWS_W = _B // _NW
_CHUNK_B = 2
_CHUNK_I = _CHUNK_B * _L
_NCHUNK = _ROWS_W // _CHUNK_B
_LANE = 16
_NG = _D_ID // _LANE

_SC_PARAMS = pltpu.CompilerParams(use_tc_tiling_on_sc=False)


def _sc_course(course_idx2d, w_course):
    mesh = plsc.VectorSubcoreMesh(core_axis_name="c", subcore_axis_name="s")

    @functools.partial(
        pl.kernel,
        out_type=jax.ShapeDtypeStruct((_B // 2, 2 * _D_ID), jnp.float32),
        mesh=mesh,
        scratch_types=[
            pltpu.VMEM((_NCHUNK, _CHUNK_I), jnp.int32),
            pltpu.VMEM((_CHUNK_I, _D_ID), jnp.float32),
            pltpu.VMEM((_CHUNK_I, _D_ID), jnp.float32),
            pltpu.VMEM((_CHUNK_I, _D_ID), jnp.float32),
            pltpu.VMEM((_CHUNK_I, _D_ID), jnp.float32),
            pltpu.VMEM((_NCHUNK, 2 * _D_ID), jnp.float32),
            pltpu.SemaphoreType.DMA,
            pltpu.SemaphoreType.DMA,
            pltpu.SemaphoreType.DMA,
            pltpu.SemaphoreType.DMA,
        ],
        compiler_params=_SC_PARAMS,
    )
    def k(cidx_hbm, wc_hbm, crs_out, cidx_v, buf_a, buf_b, buf_c, buf_d,
          out_v, sem_a, sem_b, sem_c, sem_d):
        wid = lax.axis_index("s") * _NC + lax.axis_index("c")

        pltpu.sync_copy(cidx_hbm.at[pl.ds(wid * _NCHUNK, _NCHUNK)], cidx_v)

        def issue(c, buf, sem):
            pltpu.async_copy(wc_hbm.at[cidx_v.at[c]], buf, sem)

        def wait(c, buf, sem):
            pltpu.make_async_copy(wc_hbm.at[cidx_v.at[c]], buf, sem).wait()

        bufs = (buf_a, buf_b, buf_c, buf_d)
        sems = (sem_a, sem_b, sem_c, sem_d)
        nbuf = 4
        zero = jnp.zeros((_LANE,), jnp.float32)

        for b in range(nbuf):
            issue(b, bufs[b], sems[b])

        @pl.loop(0, _NCHUNK, step=nbuf)
        def _(c):
            for b in range(nbuf):
                wait(c + b, bufs[b], sems[b])
                for r in range(_CHUNK_B):
                    for g in range(_NG):
                        out_v[c + b,
                              pl.ds(r * _D_ID + g * _LANE, _LANE)] = zero

                @pl.loop(0, _L, step=10)
                def _(l, _b=b):
                    for dl in range(10):
                        vals = [
                            bufs[_b][r * _L + l + dl,
                                     pl.ds(g * _LANE, _LANE)]
                            for r in range(_CHUNK_B) for g in range(_NG)]
                        for i, (r, g) in enumerate(
                                (r, g) for r in range(_CHUNK_B)
                                for g in range(_NG)):
                            plsc.addupdate(
                                out_v.at[c + _b,
                                         pl.ds(r * _D_ID + g * _LANE,
                                               _LANE)],
                                vals[i])

                @pl.when(c + b + nbuf < _NCHUNK)
                def _(_b=b):
                    issue(c + _b + nbuf, bufs[_b], sems[_b])

        @pl.loop(0, _NCHUNK)
        def _(c):
            for h in range(2 * _NG):
                out_v[c, pl.ds(h * _LANE, _LANE)] = (
                    out_v[c, pl.ds(h * _LANE, _LANE)] * (1.0 / _L))

        pltpu.sync_copy(out_v, crs_out.at[pl.ds(wid * _NCHUNK, _NCHUNK)])

    return k(course_idx2d, w_course)


def _sc_student(jidx2d, ws128):
    mesh = plsc.VectorSubcoreMesh(core_axis_name="c", subcore_axis_name="s")

    @functools.partial(
        pl.kernel,
        out_type=jax.ShapeDtypeStruct((_B, 2 * _D_ID), jnp.float32),
        mesh=mesh,
        scratch_types=[
            pltpu.VMEM((4, 128), jnp.int32),
            pltpu.VMEM((128, 2 * _D_ID), jnp.float32),
            pltpu.SemaphoreType.DMA,
        ],
        compiler_params=_SC_PARAMS,
    )
    def k(jidx_hbm, ws_hbm, stu_out, jidx_v, srows_v, sem):
        wid = lax.axis_index("s") * _NC + lax.axis_index("c")
        pltpu.sync_copy(jidx_hbm.at[pl.ds(wid * 4, 4)], jidx_v)
        for j in range(4):
            pltpu.async_copy(ws_hbm.at[jidx_v.at[j]], srows_v, sem).wait()
            pltpu.sync_copy(
                srows_v, stu_out.at[pl.ds(wid * _ROWS_W + j * 128, 128)])

    return k(jidx2d, ws128)


_RELAY_R = 1000


def _relay_body(a_ref, b_ref, out_ref):
    out_ref[...] = jnp.concatenate([a_ref[...], b_ref[...]], axis=1)


def _tc_relayout(w_student):
    half_blocks = (_N_STU // 2) // _RELAY_R
    return pl.pallas_call(
        _relay_body,
        grid=(half_blocks,),
        in_specs=[
            pl.BlockSpec((_RELAY_R, _D_ID), lambda i: (i, 0)),
            pl.BlockSpec((_RELAY_R, _D_ID), lambda i: (i + half_blocks, 0)),
        ],
        out_specs=pl.BlockSpec((_RELAY_R, 2 * _D_ID), lambda i: (i, 0)),
        out_shape=jax.ShapeDtypeStruct((_N_STU // 2, 2 * _D_ID), jnp.float32),
    )(w_student, w_student)


_BLK = 512


def _tc_body(hist_ref, term_ref, col_ref, maj_ref, cc_ref, stu_ref, par_ref,
             wh_ref, bh_ref, wt_ref, wcol_ref, wmaj_ref, wcc_ref, bcc_ref,
             out_ref):
    hist = hist_ref[...]
    hproj = (jnp.dot(hist, wh_ref[...], preferred_element_type=jnp.float32)
             * (1.0 / _L) + bh_ref[...])

    term = term_ref[...]
    bins = lax.broadcasted_iota(jnp.int32, (1, 64), 1)
    counts = jnp.zeros((_BLK, 64), jnp.float32)
    for l in range(_L):
        counts = counts + (term[:, l:l + 1] == bins).astype(jnp.float32)
    term_mean = jnp.dot(counts, wt_ref[...],
                        preferred_element_type=jnp.float32) * (1.0 / _L)

    stu128 = stu_ref[...]
    par = par_ref[...]
    e_stu = jnp.where(par == 1, stu128[:, _D_ID:], stu128[:, :_D_ID])

    col_oh = (col_ref[...] == lax.broadcasted_iota(jnp.int32, (1, 32), 1)
              ).astype(jnp.float32)
    e_col = jnp.dot(col_oh, wcol_ref[...], preferred_element_type=jnp.float32)

    maj_oh = (maj_ref[...] == lax.broadcasted_iota(jnp.int32, (1, 256), 1)
              ).astype(jnp.float32)
    e_maj = jnp.dot(maj_oh, wmaj_ref[...], preferred_element_type=jnp.float32)

    cc = cc_ref[...]
    wcc = wcc_ref[...]
    c_proj = cc[:, 0:1] * wcc[0:1, :] + cc[:, 1:2] * wcc[1:2, :] + bcc_ref[...]

    out_ref[...] = jnp.concatenate(
        [term_mean, hproj, e_stu, e_col, e_maj, c_proj], axis=1)


def _tc_dense(hist_flat, term_idx, col2, maj2, course_cont, stu128, par2,
              wh_rep, bh2, wt_pad, wcol_pad, wmaj_pad, w_cc, bcc2):
    grid = (_B // _BLK,)
    full = lambda shape: pl.BlockSpec(shape, lambda i: (0, 0))
    blk = lambda minor: pl.BlockSpec((_BLK, minor), lambda i: (i, 0))
    return pl.pallas_call(
        _tc_body,
        grid=grid,
        in_specs=[
            blk(_L * 16),
            blk(_L),
            blk(1),
            blk(1),
            blk(2),
            blk(128),
            blk(1),
            full((_L * 16, 16)),
            full((1, 16)),
            full((64, 32)),
            full((32, 16)),
            full((256, 16)),
            full((2, 16)),
            full((1, 16)),
        ],
        out_specs=blk(160),
        out_shape=jax.ShapeDtypeStruct((_B, 160), jnp.float32),
    )(hist_flat, term_idx, col2, maj2, course_cont, stu128, par2,
      wh_rep, bh2, wt_pad, wcol_pad, wmaj_pad, w_cc, bcc2)


def kernel(student_idx, course_idx, term_idx, college_idx, major_idx,
           hist_cont, course_cont,
           W_student, W_course, W_term, W_college, W_major,
           W_hist, b_hist, W_cc, b_cc):
    cidx2 = course_idx.astype(jnp.int32).reshape(_B * _L // _CHUNK_I, _CHUNK_I)
    crs128 = _sc_course(cidx2, W_course)
    crs_mean = crs128.reshape(_B, _D_ID)

    sidx = student_idx.astype(jnp.int32)
    ws128 = _tc_relayout(W_student)
    half = _N_STU // 2
    jidx = jnp.where(sidx < half, sidx, sidx - half)
    stu128 = _sc_student(jidx.reshape(128, 128), ws128)

    hist_flat = hist_cont.reshape(_B, _L * 16)
    wh_rep = jnp.tile(W_hist, (_L, 1))
    wt_pad = jnp.zeros((64, 32), jnp.float32).at[:51].set(W_term)
    wcol_pad = jnp.zeros((32, 16), jnp.float32).at[:31].set(W_college)
    wmaj_pad = jnp.zeros((256, 16), jnp.float32).at[:201].set(W_major)
    tc = _tc_dense(hist_flat, term_idx.astype(jnp.int32),
                   college_idx.astype(jnp.int32).reshape(_B, 1),
                   major_idx.astype(jnp.int32).reshape(_B, 1),
                   course_cont, stu128,
                   (sidx >= half).astype(jnp.int32).reshape(_B, 1),
                   wh_rep, b_hist.reshape(1, 16),
                   wt_pad, wcol_pad, wmaj_pad, W_cc, b_cc.reshape(1, 16))

    return jnp.concatenate([crs_mean, tc], axis=1)


# speedup vs baseline: 1.2056x; 1.2056x over previous
import functools

import jax
import jax.numpy as jnp
from jax import lax
from jax.experimental import pallas as pl
from jax.experimental.pallas import tpu as pltpu
from jax.experimental.pallas import tpu_sc as plsc

_B = 16384
_L = 50
_D_ID = 64
_N_STU = 1000000
_NC = 2
_NS = 16
_NW = _NC * _NS
_ROWS_W = _B // _NW
_CHUNK_B = 2
_CHUNK_I = _CHUNK_B * _L
_NCHUNK = _ROWS_W // _CHUNK_B
_LANE = 16
_NG = _D_ID // _LANE

_SC_PARAMS = pltpu.CompilerParams(use_tc_tiling_on_sc=False)


def _sc_course(course_idx2d, w_course):
    mesh = plsc.VectorSubcoreMesh(core_axis_name="c", subcore_axis_name="s")

    @functools.partial(
        pl.kernel,
        out_type=jax.ShapeDtypeStruct((_B // 2, 2 * _D_ID), jnp.float32),
        mesh=mesh,
        scratch_types=[
            pltpu.VMEM((_NCHUNK, _CHUNK_I), jnp.int32),
            pltpu.VMEM((_CHUNK_I, _D_ID), jnp.float32),
            pltpu.VMEM((_CHUNK_I, _D_ID), jnp.float32),
            pltpu.VMEM((_CHUNK_I, _D_ID), jnp.float32),
            pltpu.VMEM((_CHUNK_I, _D_ID), jnp.float32),
            pltpu.VMEM((_NCHUNK, 2 * _D_ID), jnp.float32),
            pltpu.SemaphoreType.DMA,
            pltpu.SemaphoreType.DMA,
            pltpu.SemaphoreType.DMA,
            pltpu.SemaphoreType.DMA,
        ],
        compiler_params=_SC_PARAMS,
    )
    def k(cidx_hbm, wc_hbm, crs_out, cidx_v, buf_a, buf_b, buf_c, buf_d,
          out_v, sem_a, sem_b, sem_c, sem_d):
        wid = lax.axis_index("s") * _NC + lax.axis_index("c")

        pltpu.sync_copy(cidx_hbm.at[pl.ds(wid * _NCHUNK, _NCHUNK)], cidx_v)

        def issue(c, buf, sem):
            pltpu.async_copy(wc_hbm.at[cidx_v.at[c]], buf, sem)

        def wait(c, buf, sem):
            pltpu.make_async_copy(wc_hbm.at[cidx_v.at[c]], buf, sem).wait()

        bufs = (buf_a, buf_b, buf_c, buf_d)
        sems = (sem_a, sem_b, sem_c, sem_d)
        nbuf = 4
        zero = jnp.zeros((_LANE,), jnp.float32)

        for b in range(nbuf):
            issue(b, bufs[b], sems[b])

        @pl.loop(0, _NCHUNK, step=nbuf)
        def _(c):
            for b in range(nbuf):
                wait(c + b, bufs[b], sems[b])
                for r in range(_CHUNK_B):
                    for g in range(_NG):
                        out_v[c + b,
                              pl.ds(r * _D_ID + g * _LANE, _LANE)] = zero

                @pl.loop(0, _L, step=10)
                def _(l, _b=b):
                    for dl in range(10):
                        vals = [
                            bufs[_b][r * _L + l + dl,
                                     pl.ds(g * _LANE, _LANE)]
                            for r in range(_CHUNK_B) for g in range(_NG)]
                        for i, (r, g) in enumerate(
                                (r, g) for r in range(_CHUNK_B)
                                for g in range(_NG)):
                            plsc.addupdate(
                                out_v.at[c + _b,
                                         pl.ds(r * _D_ID + g * _LANE,
                                               _LANE)],
                                vals[i])

                @pl.when(c + b + nbuf < _NCHUNK)
                def _(_b=b):
                    issue(c + _b + nbuf, bufs[_b], sems[_b])

        @pl.loop(0, _NCHUNK)
        def _(c):
            for h in range(2 * _NG):
                out_v[c, pl.ds(h * _LANE, _LANE)] = (
                    out_v[c, pl.ds(h * _LANE, _LANE)] * (1.0 / _L))

        pltpu.sync_copy(out_v, crs_out.at[pl.ds(wid * _NCHUNK, _NCHUNK)])

    return k(course_idx2d, w_course)


def _sc_student(jidx2d, ws128):
    mesh = plsc.VectorSubcoreMesh(core_axis_name="c", subcore_axis_name="s")

    @functools.partial(
        pl.kernel,
        out_type=jax.ShapeDtypeStruct((_B, 2 * _D_ID), jnp.float32),
        mesh=mesh,
        scratch_types=[
            pltpu.VMEM((4, 128), jnp.int32),
            pltpu.VMEM((128, 2 * _D_ID), jnp.float32),
            pltpu.SemaphoreType.DMA,
        ],
        compiler_params=_SC_PARAMS,
    )
    def k(jidx_hbm, ws_hbm, stu_out, jidx_v, srows_v, sem):
        wid = lax.axis_index("s") * _NC + lax.axis_index("c")
        pltpu.sync_copy(jidx_hbm.at[pl.ds(wid * 4, 4)], jidx_v)
        for j in range(4):
            pltpu.async_copy(ws_hbm.at[jidx_v.at[j]], srows_v, sem).wait()
            pltpu.sync_copy(
                srows_v, stu_out.at[pl.ds(wid * _ROWS_W + j * 128, 128)])

    return k(jidx2d, ws128)


_RELAY_R = 5000


def _relay_body(a_ref, b_ref, out_ref):
    out_ref[...] = jnp.concatenate([a_ref[...], b_ref[...]], axis=1)


def _tc_relayout(w_student):
    half_blocks = (_N_STU // 2) // _RELAY_R
    return pl.pallas_call(
        _relay_body,
        grid=(half_blocks,),
        in_specs=[
            pl.BlockSpec((_RELAY_R, _D_ID), lambda i: (i, 0)),
            pl.BlockSpec((_RELAY_R, _D_ID), lambda i: (i + half_blocks, 0)),
        ],
        out_specs=pl.BlockSpec((_RELAY_R, 2 * _D_ID), lambda i: (i, 0)),
        out_shape=jax.ShapeDtypeStruct((_N_STU // 2, 2 * _D_ID), jnp.float32),
    )(w_student, w_student)


_BLK = 512


def _tc_body(hist_ref, term_ref, col_ref, maj_ref, cc_ref, stu_ref, par_ref,
             wh_ref, bh_ref, wt_ref, wcol_ref, wmaj_ref, wcc_ref, bcc_ref,
             out_ref):
    hist = hist_ref[...]
    hproj = (jnp.dot(hist, wh_ref[...], preferred_element_type=jnp.float32)
             * (1.0 / _L) + bh_ref[...])

    term = term_ref[...]
    bins = lax.broadcasted_iota(jnp.int32, (1, 64), 1)
    counts = jnp.zeros((_BLK, 64), jnp.float32)
    for l in range(_L):
        counts = counts + (term[:, l:l + 1] == bins).astype(jnp.float32)
    term_mean = jnp.dot(counts, wt_ref[...],
                        preferred_element_type=jnp.float32) * (1.0 / _L)

    stu128 = stu_ref[...]
    par = par_ref[...]
    e_stu = jnp.where(par == 1, stu128[:, _D_ID:], stu128[:, :_D_ID])

    col_oh = (col_ref[...] == lax.broadcasted_iota(jnp.int32, (1, 32), 1)
              ).astype(jnp.float32)
    e_col = jnp.dot(col_oh, wcol_ref[...], preferred_element_type=jnp.float32)

    maj_oh = (maj_ref[...] == lax.broadcasted_iota(jnp.int32, (1, 256), 1)
              ).astype(jnp.float32)
    e_maj = jnp.dot(maj_oh, wmaj_ref[...], preferred_element_type=jnp.float32)

    cc = cc_ref[...]
    wcc = wcc_ref[...]
    c_proj = cc[:, 0:1] * wcc[0:1, :] + cc[:, 1:2] * wcc[1:2, :] + bcc_ref[...]

    out_ref[...] = jnp.concatenate(
        [term_mean, hproj, e_stu, e_col, e_maj, c_proj], axis=1)


def _tc_dense(hist_flat, term_idx, col2, maj2, course_cont, stu128, par2,
              wh_rep, bh2, wt_pad, wcol_pad, wmaj_pad, w_cc, bcc2):
    grid = (_B // _BLK,)
    full = lambda shape: pl.BlockSpec(shape, lambda i: (0, 0))
    blk = lambda minor: pl.BlockSpec((_BLK, minor), lambda i: (i, 0))
    return pl.pallas_call(
        _tc_body,
        grid=grid,
        in_specs=[
            blk(_L * 16),
            blk(_L),
            blk(1),
            blk(1),
            blk(2),
            blk(128),
            blk(1),
            full((_L * 16, 16)),
            full((1, 16)),
            full((64, 32)),
            full((32, 16)),
            full((256, 16)),
            full((2, 16)),
            full((1, 16)),
        ],
        out_specs=blk(160),
        out_shape=jax.ShapeDtypeStruct((_B, 160), jnp.float32),
    )(hist_flat, term_idx, col2, maj2, course_cont, stu128, par2,
      wh_rep, bh2, wt_pad, wcol_pad, wmaj_pad, w_cc, bcc2)


def kernel(student_idx, course_idx, term_idx, college_idx, major_idx,
           hist_cont, course_cont,
           W_student, W_course, W_term, W_college, W_major,
           W_hist, b_hist, W_cc, b_cc):
    cidx2 = course_idx.astype(jnp.int32).reshape(_B * _L // _CHUNK_I, _CHUNK_I)
    crs128 = _sc_course(cidx2, W_course)
    crs_mean = crs128.reshape(_B, _D_ID)

    sidx = student_idx.astype(jnp.int32)
    ws128 = _tc_relayout(W_student)
    half = _N_STU // 2
    jidx = jnp.where(sidx < half, sidx, sidx - half)
    stu128 = _sc_student(jidx.reshape(128, 128), ws128)

    hist_flat = hist_cont.reshape(_B, _L * 16)
    wh_rep = jnp.tile(W_hist, (_L, 1))
    wt_pad = jnp.zeros((64, 32), jnp.float32).at[:51].set(W_term)
    wcol_pad = jnp.zeros((32, 16), jnp.float32).at[:31].set(W_college)
    wmaj_pad = jnp.zeros((256, 16), jnp.float32).at[:201].set(W_major)
    tc = _tc_dense(hist_flat, term_idx.astype(jnp.int32),
                   college_idx.astype(jnp.int32).reshape(_B, 1),
                   major_idx.astype(jnp.int32).reshape(_B, 1),
                   course_cont, stu128,
                   (sidx >= half).astype(jnp.int32).reshape(_B, 1),
                   wh_rep, b_hist.reshape(1, 16),
                   wt_pad, wcol_pad, wmaj_pad, W_cc, b_cc.reshape(1, 16))

    return jnp.concatenate([crs_mean, tc], axis=1)


# speedup vs baseline: 1.2072x; 1.0014x over previous
import functools

import jax
import jax.numpy as jnp
from jax import lax
from jax.experimental import pallas as pl
from jax.experimental.pallas import tpu as pltpu
from jax.experimental.pallas import tpu_sc as plsc

_B = 16384
_L = 50
_D_ID = 64
_N_STU = 1000000
_NC = 2
_NS = 16
_NW = _NC * _NS
_ROWS_W = _B // _NW
_CHUNK_B = 2
_CHUNK_I = _CHUNK_B * _L
_NCHUNK = _ROWS_W // _CHUNK_B
_LANE = 16
_NG = _D_ID // _LANE

_SC_PARAMS = pltpu.CompilerParams(use_tc_tiling_on_sc=False)


def _sc_course(course_idx2d, w_course):
    mesh = plsc.VectorSubcoreMesh(core_axis_name="c", subcore_axis_name="s")

    @functools.partial(
        pl.kernel,
        out_type=jax.ShapeDtypeStruct((_B // 2, 2 * _D_ID), jnp.float32),
        mesh=mesh,
        scratch_types=[
            pltpu.VMEM((_NCHUNK, _CHUNK_I), jnp.int32),
            pltpu.VMEM((_CHUNK_I, _D_ID), jnp.float32),
            pltpu.VMEM((_CHUNK_I, _D_ID), jnp.float32),
            pltpu.VMEM((_CHUNK_I, _D_ID), jnp.float32),
            pltpu.VMEM((_CHUNK_I, _D_ID), jnp.float32),
            pltpu.VMEM((_NCHUNK, 2 * _D_ID), jnp.float32),
            pltpu.SemaphoreType.DMA,
            pltpu.SemaphoreType.DMA,
            pltpu.SemaphoreType.DMA,
            pltpu.SemaphoreType.DMA,
        ],
        compiler_params=_SC_PARAMS,
    )
    def k(cidx_hbm, wc_hbm, crs_out, cidx_v, buf_a, buf_b, buf_c, buf_d,
          out_v, sem_a, sem_b, sem_c, sem_d):
        wid = lax.axis_index("s") * _NC + lax.axis_index("c")

        pltpu.sync_copy(cidx_hbm.at[pl.ds(wid * _NCHUNK, _NCHUNK)], cidx_v)

        def issue(c, buf, sem):
            pltpu.async_copy(wc_hbm.at[cidx_v.at[c]], buf, sem)

        def wait(c, buf, sem):
            pltpu.make_async_copy(wc_hbm.at[cidx_v.at[c]], buf, sem).wait()

        bufs = (buf_a, buf_b, buf_c, buf_d)
        sems = (sem_a, sem_b, sem_c, sem_d)
        nbuf = 4
        zero = jnp.zeros((_LANE,), jnp.float32)

        for b in range(nbuf):
            issue(b, bufs[b], sems[b])

        @pl.loop(0, _NCHUNK, step=nbuf)
        def _(c):
            for b in range(nbuf):
                wait(c + b, bufs[b], sems[b])
                for r in range(_CHUNK_B):
                    for g in range(_NG):
                        out_v[c + b,
                              pl.ds(r * _D_ID + g * _LANE, _LANE)] = zero

                @pl.loop(0, _L, step=10)
                def _(l, _b=b):
                    for dl in range(10):
                        vals = [
                            bufs[_b][r * _L + l + dl,
                                     pl.ds(g * _LANE, _LANE)]
                            for r in range(_CHUNK_B) for g in range(_NG)]
                        for i, (r, g) in enumerate(
                                (r, g) for r in range(_CHUNK_B)
                                for g in range(_NG)):
                            plsc.addupdate(
                                out_v.at[c + _b,
                                         pl.ds(r * _D_ID + g * _LANE,
                                               _LANE)],
                                vals[i])

                @pl.when(c + b + nbuf < _NCHUNK)
                def _(_b=b):
                    issue(c + _b + nbuf, bufs[_b], sems[_b])

        @pl.loop(0, _NCHUNK)
        def _(c):
            for h in range(2 * _NG):
                out_v[c, pl.ds(h * _LANE, _LANE)] = (
                    out_v[c, pl.ds(h * _LANE, _LANE)] * (1.0 / _L))

        pltpu.sync_copy(out_v, crs_out.at[pl.ds(wid * _NCHUNK, _NCHUNK)])

    return k(course_idx2d, w_course)


def _sc_student(jidx2d, ws128):
    mesh = plsc.VectorSubcoreMesh(core_axis_name="c", subcore_axis_name="s")

    @functools.partial(
        pl.kernel,
        out_type=jax.ShapeDtypeStruct((_B, 2 * _D_ID), jnp.float32),
        mesh=mesh,
        scratch_types=[
            pltpu.VMEM((4, 128), jnp.int32),
            pltpu.VMEM((128, 2 * _D_ID), jnp.float32),
            pltpu.SemaphoreType.DMA,
        ],
        compiler_params=_SC_PARAMS,
    )
    def k(jidx_hbm, ws_hbm, stu_out, jidx_v, srows_v, sem):
        wid = lax.axis_index("s") * _NC + lax.axis_index("c")
        pltpu.sync_copy(jidx_hbm.at[pl.ds(wid * 4, 4)], jidx_v)
        for j in range(4):
            pltpu.async_copy(ws_hbm.at[jidx_v.at[j]], srows_v, sem).wait()
            pltpu.sync_copy(
                srows_v, stu_out.at[pl.ds(wid * _ROWS_W + j * 128, 128)])

    return k(jidx2d, ws128)


_BLK = 512


def _tc_body(hist_ref, term_ref, col_ref, maj_ref, cc_ref, stu_ref, par_ref,
             wh_ref, bh_ref, wt_ref, wcol_ref, wmaj_ref, wcc_ref, bcc_ref,
             out_ref):
    hist = hist_ref[...]
    hproj = (jnp.dot(hist, wh_ref[...], preferred_element_type=jnp.float32)
             * (1.0 / _L) + bh_ref[...])

    term = term_ref[...]
    bins = lax.broadcasted_iota(jnp.int32, (1, 64), 1)
    counts = jnp.zeros((_BLK, 64), jnp.float32)
    for l in range(_L):
        counts = counts + (term[:, l:l + 1] == bins).astype(jnp.float32)
    term_mean = jnp.dot(counts, wt_ref[...],
                        preferred_element_type=jnp.float32) * (1.0 / _L)

    stu128 = stu_ref[...]
    par = par_ref[...]
    e_stu = jnp.where(par == 1, stu128[:, _D_ID:], stu128[:, :_D_ID])

    col_oh = (col_ref[...] == lax.broadcasted_iota(jnp.int32, (1, 32), 1)
              ).astype(jnp.float32)
    e_col = jnp.dot(col_oh, wcol_ref[...], preferred_element_type=jnp.float32)

    maj_oh = (maj_ref[...] == lax.broadcasted_iota(jnp.int32, (1, 256), 1)
              ).astype(jnp.float32)
    e_maj = jnp.dot(maj_oh, wmaj_ref[...], preferred_element_type=jnp.float32)

    cc = cc_ref[...]
    wcc = wcc_ref[...]
    c_proj = cc[:, 0:1] * wcc[0:1, :] + cc[:, 1:2] * wcc[1:2, :] + bcc_ref[...]

    out_ref[...] = jnp.concatenate(
        [term_mean, hproj, e_stu, e_col, e_maj, c_proj], axis=1)


def _tc_dense(hist_flat, term_idx, col2, maj2, course_cont, stu128, par2,
              wh_rep, bh2, wt_pad, wcol_pad, wmaj_pad, w_cc, bcc2):
    grid = (_B // _BLK,)
    full = lambda shape: pl.BlockSpec(shape, lambda i: (0, 0))
    blk = lambda minor: pl.BlockSpec((_BLK, minor), lambda i: (i, 0))
    return pl.pallas_call(
        _tc_body,
        grid=grid,
        in_specs=[
            blk(_L * 16),
            blk(_L),
            blk(1),
            blk(1),
            blk(2),
            blk(128),
            blk(1),
            full((_L * 16, 16)),
            full((1, 16)),
            full((64, 32)),
            full((32, 16)),
            full((256, 16)),
            full((2, 16)),
            full((1, 16)),
        ],
        out_specs=blk(160),
        out_shape=jax.ShapeDtypeStruct((_B, 160), jnp.float32),
    )(hist_flat, term_idx, col2, maj2, course_cont, stu128, par2,
      wh_rep, bh2, wt_pad, wcol_pad, wmaj_pad, w_cc, bcc2)


def kernel(student_idx, course_idx, term_idx, college_idx, major_idx,
           hist_cont, course_cont,
           W_student, W_course, W_term, W_college, W_major,
           W_hist, b_hist, W_cc, b_cc):
    cidx2 = course_idx.astype(jnp.int32).reshape(_B * _L // _CHUNK_I, _CHUNK_I)
    crs128 = _sc_course(cidx2, W_course)
    crs_mean = crs128.reshape(_B, _D_ID)

    sidx = student_idx.astype(jnp.int32)
    ws128 = W_student[:_N_STU].reshape(_N_STU // 2, 2 * _D_ID)
    stu128 = _sc_student((sidx // 2).reshape(128, 128), ws128)

    hist_flat = hist_cont.reshape(_B, _L * 16)
    wh_rep = jnp.tile(W_hist, (_L, 1))
    wt_pad = jnp.zeros((64, 32), jnp.float32).at[:51].set(W_term)
    wcol_pad = jnp.zeros((32, 16), jnp.float32).at[:31].set(W_college)
    wmaj_pad = jnp.zeros((256, 16), jnp.float32).at[:201].set(W_major)
    tc = _tc_dense(hist_flat, term_idx.astype(jnp.int32),
                   college_idx.astype(jnp.int32).reshape(_B, 1),
                   major_idx.astype(jnp.int32).reshape(_B, 1),
                   course_cont, stu128, (sidx % 2).reshape(_B, 1),
                   wh_rep, b_hist.reshape(1, 16),
                   wt_pad, wcol_pad, wmaj_pad, W_cc, b_cc.reshape(1, 16))

    return jnp.concatenate([crs_mean, tc], axis=1)
